# Initial kernel scaffold; baseline (speedup 1.0000x reference)
import jax
import jax.numpy as jnp
from jax.experimental import pallas as pl


def kernel(x, edge_index, W0, a_src0, a_dst0, b0, Rw0, Rb0, W1, a_src1, a_dst1, b1, Rw1, Rb1):
    raise NotImplementedError("write your pallas kernel here")


# speedup vs baseline: 34.1574x; 34.1574x over previous
import jax
import jax.numpy as jnp
from jax import lax
from jax.experimental import pallas as pl
from jax.experimental.pallas import tpu as pltpu
from jax.experimental.pallas import tpu_sc as plsc

N = 10000
D = 128
E = 320000
NW = 32
EPW = E // NW
CH = 80
NCH = EPW // CH
ZR = 624
ZB = 208
BN = 1000


def _proj_body(x_ref, w_ref, as_ref, ad_ref, rw_ref, rb_ref,
               h_ref, asrc_ref, adst_ref, res_ref):
    xb = x_ref[...]
    h = jnp.dot(xb, w_ref[...], preferred_element_type=jnp.float32)
    h_ref[...] = h
    asrc_ref[...] = jnp.dot(h, as_ref[...], preferred_element_type=jnp.float32)
    adst_ref[...] = jnp.dot(h, ad_ref[...], preferred_element_type=jnp.float32)
    res_ref[...] = jnp.dot(xb, rw_ref[...], preferred_element_type=jnp.float32) + rb_ref[...]


def _tc_proj(x, w, a_s, a_d, rw, rb):
    return pl.pallas_call(
        _proj_body,
        grid=(N // BN,),
        in_specs=[
            pl.BlockSpec((BN, D), lambda i: (i, 0)),
            pl.BlockSpec((D, D), lambda i: (0, 0)),
            pl.BlockSpec((D, 1), lambda i: (0, 0)),
            pl.BlockSpec((D, 1), lambda i: (0, 0)),
            pl.BlockSpec((D, D), lambda i: (0, 0)),
            pl.BlockSpec((1, D), lambda i: (0, 0)),
        ],
        out_specs=[
            pl.BlockSpec((BN, D), lambda i: (i, 0)),
            pl.BlockSpec((BN, 1), lambda i: (i, 0)),
            pl.BlockSpec((BN, 1), lambda i: (i, 0)),
            pl.BlockSpec((BN, D), lambda i: (i, 0)),
        ],
        out_shape=[
            jax.ShapeDtypeStruct((N, D), jnp.float32),
            jax.ShapeDtypeStruct((N, 1), jnp.float32),
            jax.ShapeDtypeStruct((N, 1), jnp.float32),
            jax.ShapeDtypeStruct((N, D), jnp.float32),
        ],
    )(x, w, a_s, a_d, rw, rb)


def _comb_proj_body(aggp_ref, denp_ref, res_ref, w_ref, as_ref, ad_ref, rw_ref, rb_ref,
                    h_ref, asrc_ref, adst_ref, res2_ref):
    den = jnp.sum(denp_ref[0], axis=1)
    v = (aggp_ref[0] + aggp_ref[1]) / (den[:, None] + 1e-16) + res_ref[...]
    x1 = jnp.where(v > 0, v, jnp.exp(v) - 1.0)
    h = jnp.dot(x1, w_ref[...], preferred_element_type=jnp.float32)
    h_ref[...] = h
    asrc_ref[...] = jnp.dot(h, as_ref[...], preferred_element_type=jnp.float32)
    adst_ref[...] = jnp.dot(h, ad_ref[...], preferred_element_type=jnp.float32)
    res2_ref[...] = jnp.dot(x1, rw_ref[...], preferred_element_type=jnp.float32) + rb_ref[...]


def _tc_comb_proj(aggp, denp, res, w, a_s, a_d, rw, rb):
    return pl.pallas_call(
        _comb_proj_body,
        grid=(N // BN,),
        in_specs=[
            pl.BlockSpec((2, BN, D), lambda i: (0, i, 0)),
            pl.BlockSpec((1, BN, 2), lambda i: (i, 0, 0)),
            pl.BlockSpec((BN, D), lambda i: (i, 0)),
            pl.BlockSpec((D, D), lambda i: (0, 0)),
            pl.BlockSpec((D, 1), lambda i: (0, 0)),
            pl.BlockSpec((D, 1), lambda i: (0, 0)),
            pl.BlockSpec((D, D), lambda i: (0, 0)),
            pl.BlockSpec((1, D), lambda i: (0, 0)),
        ],
        out_specs=[
            pl.BlockSpec((BN, D), lambda i: (i, 0)),
            pl.BlockSpec((BN, 1), lambda i: (i, 0)),
            pl.BlockSpec((BN, 1), lambda i: (i, 0)),
            pl.BlockSpec((BN, D), lambda i: (i, 0)),
        ],
        out_shape=[
            jax.ShapeDtypeStruct((N, D), jnp.float32),
            jax.ShapeDtypeStruct((N, 1), jnp.float32),
            jax.ShapeDtypeStruct((N, 1), jnp.float32),
            jax.ShapeDtypeStruct((N, D), jnp.float32),
        ],
    )(aggp, denp, res, w, a_s, a_d, rw, rb)


def _final_body(aggp_ref, denp_ref, res_ref, out_ref):
    den = jnp.sum(denp_ref[0], axis=1)
    v = (aggp_ref[0] + aggp_ref[1]) / (den[:, None] + 1e-16) + res_ref[...]
    out_ref[...] = jnp.where(v > 0, v, jnp.exp(v) - 1.0)


def _tc_final(aggp, denp, res):
    return pl.pallas_call(
        _final_body,
        grid=(N // BN,),
        in_specs=[
            pl.BlockSpec((2, BN, D), lambda i: (0, i, 0)),
            pl.BlockSpec((1, BN, 2), lambda i: (i, 0, 0)),
            pl.BlockSpec((BN, D), lambda i: (i, 0)),
        ],
        out_specs=pl.BlockSpec((BN, D), lambda i: (i, 0)),
        out_shape=jax.ShapeDtypeStruct((N, D), jnp.float32),
    )(aggp, denp, res)


def _sc_edge_body(h_hbm, asrc_hbm, adst_hbm, sd_hbm,
                  aggp_hbm, denp_hbm,
                  asrc_v, adst_v, sd_v, ex_v, zden_v, rows_v,
                  agg_sh, den_sh, semg, semd, sema):
    c = lax.axis_index("c")
    s = lax.axis_index("s")
    wid = c * 16 + s
    cbase = wid * NCH

    pltpu.sync_copy(asrc_hbm, asrc_v)
    pltpu.sync_copy(adst_hbm, adst_v)

    zeros = jnp.zeros((16,), jnp.float32)

    def _zero_rows(i, carry):
        for cb in range(8):
            rows_v[0, i, pl.ds(cb * 16, 16)] = zeros
        return carry

    lax.fori_loop(0, CH, _zero_rows, 0)

    def _zero_zden(i, carry):
        zden_v[pl.ds(i * 16, 16)] = zeros
        return carry

    lax.fori_loop(0, 40, _zero_zden, 0)

    base = pl.multiple_of(s * ZR, 8)
    for r in range(7):
        pltpu.sync_copy(rows_v.at[0], agg_sh.at[pl.ds(base + r * CH, CH)])
    pltpu.sync_copy(rows_v.at[0, pl.ds(0, 64)], agg_sh.at[pl.ds(base + 7 * CH, 64)])

    @pl.when(s == 0)
    def _zero_tail():
        pltpu.sync_copy(rows_v.at[0, pl.ds(0, 16)], agg_sh.at[pl.ds(16 * ZR, 16)])

    dbase = pl.multiple_of(s * 640, 8)

    @pl.when(s < 15)
    def _zero_den_most():
        pltpu.sync_copy(zden_v, den_sh.at[pl.ds(dbase, 640)])

    @pl.when(s == 15)
    def _zero_den_last():
        pltpu.sync_copy(zden_v.at[pl.ds(0, 400)], den_sh.at[pl.ds(15 * 640, 400)])

    plsc.subcore_barrier()

    def _wait_gather(b):
        pltpu.make_async_copy(h_hbm.at[sd_v.at[b, 0]], rows_v.at[b], semg).wait()

    def _wait_scatters(b):
        pltpu.make_async_copy(ex_v.at[b], den_sh.at[sd_v.at[b, 1]], semd).wait()
        pltpu.make_async_copy(rows_v.at[b], agg_sh.at[sd_v.at[b, 1]], sema).wait()

    def _launch(j, b):
        pltpu.sync_copy(sd_hbm.at[cbase + j], sd_v.at[b])
        pltpu.async_copy(h_hbm.at[sd_v.at[b, 0]], rows_v.at[b], semg)

    def _step(j, b, first, last):
        nb = 1 - b
        _wait_gather(b)
        if not first:
            _wait_scatters(nb)
        if not last:
            _launch(j + 1, nb)

        for k in range(CH // 16):
            sidx = sd_v[b, 0, pl.ds(k * 16, 16)]
            didx = sd_v[b, 1, pl.ds(k * 16, 16)]
            e = plsc.load_gather(asrc_v, [sidx]) + plsc.load_gather(adst_v, [didx])
            e = jnp.where(e > 0, e, 0.2 * e)
            ex_v[b, pl.ds(k * 16, 16)] = jnp.exp(e)

        pltpu.async_copy(ex_v.at[b], den_sh.at[sd_v.at[b, 1]], semd, add=True)

        def _scale(eidx, carry2):
            cv = plsc.load_gather(
                ex_v, [jnp.full((16,), b, jnp.int32),
                       jnp.full((16,), eidx, jnp.int32)])
            for cb in range(8):
                rows_v[b, eidx, pl.ds(cb * 16, 16)] = (
                    rows_v[b, eidx, pl.ds(cb * 16, 16)] * cv)
            return carry2

        lax.fori_loop(0, CH, _scale, 0, unroll=8)
        pltpu.async_copy(rows_v.at[b], agg_sh.at[sd_v.at[b, 1]], sema, add=True)

    _launch(0, 0)
    _step(0, 0, True, False)

    def _pipe(t, carry):
        j0 = t * 2
        _step(j0 + 1, 1, False, False)
        _step(j0 + 2, 0, False, False)
        return carry

    lax.fori_loop(0, (NCH - 3) // 2, _pipe, 0)
    _step(NCH - 2, 1, False, False)
    _step(NCH - 1, 0, False, True)
    _wait_scatters(0)
    plsc.subcore_barrier()

    pltpu.sync_copy(agg_sh.at[pl.ds(base, ZR)], aggp_hbm.at[c, pl.ds(base, ZR)])

    @pl.when(s == 0)
    def _copy_tail():
        pltpu.sync_copy(agg_sh.at[pl.ds(16 * ZR, 16)],
                        aggp_hbm.at[c, pl.ds(16 * ZR, 16)])
        pltpu.sync_copy(den_sh, denp_hbm.at[c, 0])


def _sc_edge(h, asrc, adst, sd):
    mesh = plsc.VectorSubcoreMesh(core_axis_name="c", subcore_axis_name="s")
    return pl.kernel(
        _sc_edge_body,
        out_type=(
            jax.ShapeDtypeStruct((2, N, D), jnp.float32),
            jax.ShapeDtypeStruct((2, 1, N), jnp.float32),
        ),
        mesh=mesh,
        compiler_params=pltpu.CompilerParams(needs_layout_passes=False),
        scratch_types=[
            pltpu.VMEM((N,), jnp.float32),
            pltpu.VMEM((N,), jnp.float32),
            pltpu.VMEM((2, 2, CH), jnp.int32),
            pltpu.VMEM((2, CH), jnp.float32),
            pltpu.VMEM((640,), jnp.float32),
            pltpu.VMEM((2, CH, D), jnp.float32),
            pltpu.VMEM_SHARED((N, D), jnp.float32),
            pltpu.VMEM_SHARED((N,), jnp.float32),
            pltpu.SemaphoreType.DMA,
            pltpu.SemaphoreType.DMA,
            pltpu.SemaphoreType.DMA,
        ],
    )(h, asrc, adst, sd)


def kernel(x, edge_index, W0, a_src0, a_dst0, b0, Rw0, Rb0,
           W1, a_src1, a_dst1, b1, Rw1, Rb1):
    sd = jnp.stack(
        [edge_index[0].reshape(NW * NCH, CH),
         edge_index[1].reshape(NW * NCH, CH)], axis=1)

    h0, as0, ad0, res0 = _tc_proj(
        x, W0, a_src0[:, None], a_dst0[:, None], Rw0, (Rb0 + b0)[None, :])
    aggp0, denp0 = _sc_edge(h0, as0.reshape(N), ad0.reshape(N), sd)

    h1, as1, ad1, res1 = _tc_comb_proj(
        aggp0, denp0.reshape(2, N).T.reshape(N // BN, BN, 2), res0, W1,
        a_src1[:, None], a_dst1[:, None], Rw1, (Rb1 + b1)[None, :])
    aggp1, denp1 = _sc_edge(h1, as1.reshape(N), ad1.reshape(N), sd)

    return _tc_final(aggp1, denp1.reshape(2, N).T.reshape(N // BN, BN, 2), res1)
